# baseline (speedup 1.0000x reference)
import numpy as np
import jax
import jax.numpy as jnp
from jax import lax
from jax.experimental import pallas as pl
from jax.experimental.pallas import tpu as pltpu
from jax.experimental.pallas import tpu_sc as plsc

_N = 262144
_NB = 4
_L = 8
_F = 2
_T = 2 ** 17
_RES = [int(np.floor(16 * (1.5 ** l))) for l in range(_L)]
_P1 = np.uint32(2654435761)
_P2 = np.uint32(805459861)
_DH = 64

_NC = 2
_NS = 16
_NW = _NC * _NS
_C = 128


def _make_sc_encode(n, c, interpret=False):
    per_w = n // _NW
    nchunk = per_w // c
    g_per_c = c // 16
    nrow = _L * 8 * _F

    def body(x_hbm, bi_hbm, tabf_hbm, h_hbm, x_v, bi_v, idx_v, vals_v, h_v,
             sem):
        wid = lax.axis_index("s") * _NC + lax.axis_index("c")
        base_w = wid * per_w
        iota = lax.iota(jnp.int32, 16)

        def chunk_body(k, carry):
            base = base_w + k * c
            pltpu.sync_copy(x_hbm.at[pl.ds(base * 3, c * 3)], x_v)
            pltpu.sync_copy(bi_hbm.at[pl.ds(base, c)], bi_v)

            def idx_body(g, c2):
                o = g * 16
                i3 = (o + iota) * 3
                xv = [plsc.load_gather(x_v, [i3 + d]) for d in range(3)]
                bv = bi_v[pl.ds(o, 16)]
                for l in range(_L):
                    r1 = np.float32(_RES[l] - 1)
                    x0 = [(xv[d] * r1).astype(jnp.int32).astype(jnp.uint32)
                          for d in range(3)]
                    a0 = x0[0]
                    b0 = a0 + jnp.uint32(1)
                    a1 = x0[1] * _P1
                    b1 = a1 + _P1
                    a2 = x0[2] * _P2
                    b2 = a2 + _P2
                    lb = (bv + jnp.int32(l * _NB)) << 17
                    for cc in range(8):
                        hh = ((b0 if cc & 1 else a0)
                              ^ (b1 if cc & 2 else a1)
                              ^ (b2 if cc & 4 else a2)) & jnp.uint32(_T - 1)
                        e0 = (hh.astype(jnp.int32) + lb) * 2
                        idx_v[2 * (l * 8 + cc), pl.ds(o, 16)] = e0
                        idx_v[2 * (l * 8 + cc) + 1, pl.ds(o, 16)] = e0 + 1
                return c2

            lax.fori_loop(0, g_per_c, idx_body, 0)

            def gather_body(jg, c2):
                for jj in range(8):
                    j = jg * 8 + jj
                    pltpu.async_copy(
                        tabf_hbm.at[idx_v.at[j]], vals_v.at[j], sem)
                for jj in range(8):
                    j = jg * 8 + jj
                    pltpu.make_async_copy(
                        tabf_hbm.at[idx_v.at[j]], vals_v.at[j], sem).wait()
                return c2

            lax.fori_loop(0, nrow // 8, gather_body, 0)

            def acc_body(g, c2):
                o = g * 16
                i3 = (o + iota) * 3
                xv = [plsc.load_gather(x_v, [i3 + d]) for d in range(3)]
                for l in range(_L):
                    r1 = np.float32(_RES[l] - 1)
                    xs = [xv[d] * r1 for d in range(3)]
                    w = [xs[d] - xs[d].astype(jnp.int32).astype(jnp.float32)
                         for d in range(3)]
                    u = [np.float32(1.0) - w[d] for d in range(3)]
                    acc0 = jnp.zeros((16,), jnp.float32)
                    acc1 = jnp.zeros((16,), jnp.float32)
                    for cc in range(8):
                        wc = ((w[0] if cc & 1 else u[0])
                              * (w[1] if cc & 2 else u[1])
                              * (w[2] if cc & 4 else u[2]))
                        r0 = jnp.full((16,), 2 * (l * 8 + cc), jnp.int32)
                        f0 = plsc.load_gather(vals_v, [r0, o + iota])
                        f1 = plsc.load_gather(vals_v, [r0 + 1, o + iota])
                        acc0 = acc0 + wc * f0
                        acc1 = acc1 + wc * f1
                    plsc.store_scatter(
                        h_v, [o + iota, jnp.full((16,), 2 * l, jnp.int32)],
                        acc0)
                    plsc.store_scatter(
                        h_v, [o + iota, jnp.full((16,), 2 * l + 1, jnp.int32)],
                        acc1)
                return c2

            lax.fori_loop(0, g_per_c, acc_body, 0)

            pltpu.sync_copy(h_v, h_hbm.at[pl.ds(base, c)])
            return carry

        lax.fori_loop(0, nchunk, chunk_body, 0)

    return pl.kernel(
        body,
        out_type=jax.ShapeDtypeStruct((n, 16), jnp.float32),
        mesh=plsc.VectorSubcoreMesh(
            core_axis_name="c", subcore_axis_name="s",
            num_cores=_NC, num_subcores=_NS),
        compiler_params=pltpu.CompilerParams(
            needs_layout_passes=False, use_tc_tiling_on_sc=False),
        scratch_types=[
            pltpu.VMEM((3 * c,), jnp.float32),
            pltpu.VMEM((c,), jnp.int32),
            pltpu.VMEM((nrow, c), jnp.int32),
            pltpu.VMEM((nrow, c), jnp.float32),
            pltpu.VMEM((c, 16), jnp.float32),
            pltpu.SemaphoreType.DMA,
        ],
        interpret=interpret,
    )


_sc_encode = _make_sc_encode(_N, _C)


def _mlp_body(h_ref, w1_ref, b1_ref, w2_ref, b2_ref, w3_ref, b3_ref, sdf_ref):
    h = h_ref[...]
    z = jnp.maximum(
        jnp.dot(h, w1_ref[...], preferred_element_type=jnp.float32)
        + b1_ref[...], 0.0)
    z = jnp.maximum(
        jnp.dot(z, w2_ref[...], preferred_element_type=jnp.float32)
        + b2_ref[...], 0.0)
    sdf_ref[...] = (
        jnp.dot(z, w3_ref[...], preferred_element_type=jnp.float32)
        + b3_ref[...])


def _mlp(h, W1, b1, W2, b2, W3, b3):
    bm = 4096
    wspec = lambda shape: pl.BlockSpec(shape, lambda i: (0, 0))
    return pl.pallas_call(
        _mlp_body,
        grid=(_N // bm,),
        in_specs=[
            pl.BlockSpec((bm, 16), lambda i: (i, 0)),
            wspec((16, _DH)), wspec((1, _DH)),
            wspec((_DH, _DH)), wspec((1, _DH)),
            wspec((_DH, 1)), wspec((1, 1)),
        ],
        out_specs=pl.BlockSpec((bm, 1), lambda i: (i, 0)),
        out_shape=jax.ShapeDtypeStruct((_N, 1), jnp.float32),
    )(h, W1, b1, W2, b2, W3, b3)


def kernel(x, block_inds, tables, W1, b1, W2, b2, W3, b3):
    x_flat = x.reshape(_N * 3)
    tabf = tables.reshape(_L * _NB * _T * _F)
    h = _sc_encode(x_flat, block_inds, tabf)
    sdf2 = _mlp(h, W1, b1.reshape(1, _DH), W2, b2.reshape(1, _DH),
                W3, b3.reshape(1, 1))
    return sdf2[:, 0], h


# speedup vs baseline: 1.0762x; 1.0762x over previous
import numpy as np
import jax
import jax.numpy as jnp
from jax import lax
from jax.experimental import pallas as pl
from jax.experimental.pallas import tpu as pltpu
from jax.experimental.pallas import tpu_sc as plsc

_N = 262144
_NB = 4
_L = 8
_F = 2
_T = 2 ** 17
_RES = [int(np.floor(16 * (1.5 ** l))) for l in range(_L)]
_P1 = np.uint32(2654435761)
_P2 = np.uint32(805459861)
_DH = 64

_NC = 2
_NS = 16
_NW = _NC * _NS
_C = 128


def _make_sc_encode(n, c, interpret=False):
    per_w = n // _NW
    nchunk = per_w // c
    g_per_c = c // 16
    nrow = _L * 8 * _F

    def body(x_hbm, bi_hbm, tabf_hbm, h_hbm, x_v, bi_v, idx_v, vals_v, h_v,
             sem):
        wid = lax.axis_index("s") * _NC + lax.axis_index("c")
        base_w = wid * per_w
        iota = lax.iota(jnp.int32, 16)

        def chunk_body(k, carry):
            base = base_w + k * c
            pltpu.sync_copy(x_hbm.at[pl.ds(base * 3, c * 3)], x_v)
            pltpu.sync_copy(bi_hbm.at[pl.ds(base, c)], bi_v)

            def idx_body(g, c2):
                o = g * 16
                i3 = (o + iota) * 3
                xv = [plsc.load_gather(x_v, [i3 + d]) for d in range(3)]
                bv = bi_v[pl.ds(o, 16)]
                for l in range(_L):
                    r1 = np.float32(_RES[l] - 1)
                    x0 = [(xv[d] * r1).astype(jnp.int32).astype(jnp.uint32)
                          for d in range(3)]
                    a0 = x0[0]
                    b0 = a0 + jnp.uint32(1)
                    a1 = x0[1] * _P1
                    b1 = a1 + _P1
                    a2 = x0[2] * _P2
                    b2 = a2 + _P2
                    lb = (bv + jnp.int32(l * _NB)) << 17
                    for cc in range(8):
                        hh = ((b0 if cc & 1 else a0)
                              ^ (b1 if cc & 2 else a1)
                              ^ (b2 if cc & 4 else a2)) & jnp.uint32(_T - 1)
                        e0 = (hh.astype(jnp.int32) + lb) * 2
                        idx_v[pl.ds(2 * (l * 8 + cc) * c + o, 16)] = e0
                        idx_v[pl.ds((2 * (l * 8 + cc) + 1) * c + o, 16)] = (
                            e0 + 1)
                return c2

            lax.fori_loop(0, g_per_c, idx_body, 0)

            pltpu.async_copy(tabf_hbm.at[idx_v], vals_v, sem).wait()

            def acc_body(g, c2):
                o = g * 16
                i3 = (o + iota) * 3
                xv = [plsc.load_gather(x_v, [i3 + d]) for d in range(3)]
                for l in range(_L):
                    r1 = np.float32(_RES[l] - 1)
                    xs = [xv[d] * r1 for d in range(3)]
                    w = [xs[d] - xs[d].astype(jnp.int32).astype(jnp.float32)
                         for d in range(3)]
                    u = [np.float32(1.0) - w[d] for d in range(3)]
                    acc0 = jnp.zeros((16,), jnp.float32)
                    acc1 = jnp.zeros((16,), jnp.float32)
                    for cc in range(8):
                        wc = ((w[0] if cc & 1 else u[0])
                              * (w[1] if cc & 2 else u[1])
                              * (w[2] if cc & 4 else u[2]))
                        r0 = 2 * (l * 8 + cc) * c + o + iota
                        f0 = plsc.load_gather(vals_v, [r0])
                        f1 = plsc.load_gather(vals_v, [r0 + c])
                        acc0 = acc0 + wc * f0
                        acc1 = acc1 + wc * f1
                    plsc.store_scatter(
                        h_v, [o + iota, jnp.full((16,), 2 * l, jnp.int32)],
                        acc0)
                    plsc.store_scatter(
                        h_v, [o + iota, jnp.full((16,), 2 * l + 1, jnp.int32)],
                        acc1)
                return c2

            lax.fori_loop(0, g_per_c, acc_body, 0)

            pltpu.sync_copy(h_v, h_hbm.at[pl.ds(base, c)])
            return carry

        lax.fori_loop(0, nchunk, chunk_body, 0)

    return pl.kernel(
        body,
        out_type=jax.ShapeDtypeStruct((n, 16), jnp.float32),
        mesh=plsc.VectorSubcoreMesh(
            core_axis_name="c", subcore_axis_name="s",
            num_cores=_NC, num_subcores=_NS),
        compiler_params=pltpu.CompilerParams(
            needs_layout_passes=False, use_tc_tiling_on_sc=False),
        scratch_types=[
            pltpu.VMEM((3 * c,), jnp.float32),
            pltpu.VMEM((c,), jnp.int32),
            pltpu.VMEM((nrow * c,), jnp.int32),
            pltpu.VMEM((nrow * c,), jnp.float32),
            pltpu.VMEM((c, 16), jnp.float32),
            pltpu.SemaphoreType.DMA,
        ],
        interpret=interpret,
    )


_sc_encode = _make_sc_encode(_N, _C)


def _mlp_body(h_ref, w1_ref, b1_ref, w2_ref, b2_ref, w3_ref, b3_ref, sdf_ref):
    h = h_ref[...]
    z = jnp.maximum(
        jnp.dot(h, w1_ref[...], preferred_element_type=jnp.float32)
        + b1_ref[...], 0.0)
    z = jnp.maximum(
        jnp.dot(z, w2_ref[...], preferred_element_type=jnp.float32)
        + b2_ref[...], 0.0)
    sdf_ref[...] = (
        jnp.dot(z, w3_ref[...], preferred_element_type=jnp.float32)
        + b3_ref[...])


def _mlp(h, W1, b1, W2, b2, W3, b3):
    bm = 4096
    wspec = lambda shape: pl.BlockSpec(shape, lambda i: (0, 0))
    return pl.pallas_call(
        _mlp_body,
        grid=(_N // bm,),
        in_specs=[
            pl.BlockSpec((bm, 16), lambda i: (i, 0)),
            wspec((16, _DH)), wspec((1, _DH)),
            wspec((_DH, _DH)), wspec((1, _DH)),
            wspec((_DH, 1)), wspec((1, 1)),
        ],
        out_specs=pl.BlockSpec((bm, 1), lambda i: (i, 0)),
        out_shape=jax.ShapeDtypeStruct((_N, 1), jnp.float32),
    )(h, W1, b1, W2, b2, W3, b3)


def kernel(x, block_inds, tables, W1, b1, W2, b2, W3, b3):
    x_flat = x.reshape(_N * 3)
    tabf = tables.reshape(_L * _NB * _T * _F)
    h = _sc_encode(x_flat, block_inds, tabf)
    sdf2 = _mlp(h, W1, b1.reshape(1, _DH), W2, b2.reshape(1, _DH),
                W3, b3.reshape(1, 1))
    return sdf2[:, 0], h


# speedup vs baseline: 5.9014x; 5.4833x over previous
import numpy as np
import jax
import jax.numpy as jnp
from jax import lax
from jax.experimental import pallas as pl
from jax.experimental.pallas import tpu as pltpu
from jax.experimental.pallas import tpu_sc as plsc

_N = 262144
_NB = 4
_L = 8
_F = 2
_T = 2 ** 17
_RES = [int(np.floor(16 * (1.5 ** l))) for l in range(_L)]
_P1 = np.uint32(2654435761)
_P2 = np.uint32(805459861)
_DH = 64

_NC = 2
_NS = 16
_NW = _NC * _NS
_C = 128


def _make_sc_encode(n, c, interpret=False):
    per_w = n // _NW
    nchunk = per_w // c
    g_per_c = c // 16
    nrow = _L * 8

    def body(x_hbm, bi_hbm, tabf_hbm, h_hbm, x_v, bi_v, idx_v, vals_v, h_v,
             sem):
        wid = lax.axis_index("s") * _NC + lax.axis_index("c")
        base_w = wid * per_w
        iota = lax.iota(jnp.int32, 16)

        def chunk_body(k, carry):
            base = base_w + k * c
            pltpu.sync_copy(x_hbm.at[pl.ds(base * 3, c * 3)], x_v)
            pltpu.sync_copy(bi_hbm.at[pl.ds(base, c)], bi_v)

            def idx_body(g, c2):
                o = g * 16
                i3 = (o + iota) * 3
                xv = [plsc.load_gather(x_v, [i3 + d]) for d in range(3)]
                bv = bi_v[pl.ds(o, 16)]
                for l in range(_L):
                    r1 = np.float32(_RES[l] - 1)
                    x0 = [(xv[d] * r1).astype(jnp.int32).astype(jnp.uint32)
                          for d in range(3)]
                    a0 = x0[0]
                    b0 = a0 + jnp.uint32(1)
                    a1 = x0[1] * _P1
                    b1 = a1 + _P1
                    a2 = x0[2] * _P2
                    b2 = a2 + _P2
                    lb = (bv + jnp.int32(l * _NB)) << 17
                    for cc in range(8):
                        hh = ((b0 if cc & 1 else a0)
                              ^ (b1 if cc & 2 else a1)
                              ^ (b2 if cc & 4 else a2)) & jnp.uint32(_T - 1)
                        e0 = hh.astype(jnp.int32) + lb
                        idx_v[pl.ds((l * 8 + cc) * c + o, 16)] = e0
                return c2

            lax.fori_loop(0, g_per_c, idx_body, 0)

            pltpu.async_copy(tabf_hbm.at[idx_v], vals_v, sem).wait()

            def acc_body(g, c2):
                o = g * 16
                i3 = (o + iota) * 3
                xv = [plsc.load_gather(x_v, [i3 + d]) for d in range(3)]
                for l in range(_L):
                    r1 = np.float32(_RES[l] - 1)
                    xs = [xv[d] * r1 for d in range(3)]
                    w = [xs[d] - xs[d].astype(jnp.int32).astype(jnp.float32)
                         for d in range(3)]
                    u = [np.float32(1.0) - w[d] for d in range(3)]
                    acc0 = jnp.zeros((16,), jnp.float32)
                    acc1 = jnp.zeros((16,), jnp.float32)
                    for cc in range(8):
                        wc = ((w[0] if cc & 1 else u[0])
                              * (w[1] if cc & 2 else u[1])
                              * (w[2] if cc & 4 else u[2]))
                        r0 = (l * 8 + cc) * c + o + iota
                        v = plsc.load_gather(vals_v, [r0])
                        f0 = lax.bitcast_convert_type(
                            v << 16, jnp.float32)
                        f1 = lax.bitcast_convert_type(
                            v & jnp.int32(-65536), jnp.float32)
                        acc0 = acc0 + wc * f0
                        acc1 = acc1 + wc * f1
                    plsc.store_scatter(
                        h_v, [o + iota, jnp.full((16,), 2 * l, jnp.int32)],
                        acc0)
                    plsc.store_scatter(
                        h_v, [o + iota, jnp.full((16,), 2 * l + 1, jnp.int32)],
                        acc1)
                return c2

            lax.fori_loop(0, g_per_c, acc_body, 0)

            pltpu.sync_copy(h_v, h_hbm.at[pl.ds(base, c)])
            return carry

        lax.fori_loop(0, nchunk, chunk_body, 0)

    return pl.kernel(
        body,
        out_type=jax.ShapeDtypeStruct((n, 16), jnp.float32),
        mesh=plsc.VectorSubcoreMesh(
            core_axis_name="c", subcore_axis_name="s",
            num_cores=_NC, num_subcores=_NS),
        compiler_params=pltpu.CompilerParams(
            needs_layout_passes=False, use_tc_tiling_on_sc=False),
        scratch_types=[
            pltpu.VMEM((3 * c,), jnp.float32),
            pltpu.VMEM((c,), jnp.int32),
            pltpu.VMEM((nrow * c,), jnp.int32),
            pltpu.VMEM((nrow * c,), jnp.int32),
            pltpu.VMEM((c, 16), jnp.float32),
            pltpu.SemaphoreType.DMA,
        ],
        interpret=interpret,
    )


_sc_encode = _make_sc_encode(_N, _C)


def _mlp_body(h_ref, w1_ref, b1_ref, w2_ref, b2_ref, w3_ref, b3_ref, sdf_ref):
    h = h_ref[...]
    z = jnp.maximum(
        jnp.dot(h, w1_ref[...], preferred_element_type=jnp.float32)
        + b1_ref[...], 0.0)
    z = jnp.maximum(
        jnp.dot(z, w2_ref[...], preferred_element_type=jnp.float32)
        + b2_ref[...], 0.0)
    sdf_ref[...] = (
        jnp.dot(z, w3_ref[...], preferred_element_type=jnp.float32)
        + b3_ref[...])


def _mlp(h, W1, b1, W2, b2, W3, b3):
    bm = 4096
    wspec = lambda shape: pl.BlockSpec(shape, lambda i: (0, 0))
    return pl.pallas_call(
        _mlp_body,
        grid=(_N // bm,),
        in_specs=[
            pl.BlockSpec((bm, 16), lambda i: (i, 0)),
            wspec((16, _DH)), wspec((1, _DH)),
            wspec((_DH, _DH)), wspec((1, _DH)),
            wspec((_DH, 1)), wspec((1, 1)),
        ],
        out_specs=pl.BlockSpec((bm, 1), lambda i: (i, 0)),
        out_shape=jax.ShapeDtypeStruct((_N, 1), jnp.float32),
    )(h, W1, b1, W2, b2, W3, b3)


def kernel(x, block_inds, tables, W1, b1, W2, b2, W3, b3):
    x_flat = x.reshape(_N * 3)
    tabp = lax.bitcast_convert_type(
        tables.astype(jnp.bfloat16), jnp.int32).reshape(_L * _NB * _T)
    h = _sc_encode(x_flat, block_inds, tabp)
    sdf2 = _mlp(h, W1, b1.reshape(1, _DH), W2, b2.reshape(1, _DH),
                W3, b3.reshape(1, 1))
    return sdf2[:, 0], h


# speedup vs baseline: 6.7526x; 1.1442x over previous
import numpy as np
import jax
import jax.numpy as jnp
from jax import lax
from jax.experimental import pallas as pl
from jax.experimental.pallas import tpu as pltpu
from jax.experimental.pallas import tpu_sc as plsc

_N = 262144
_NB = 4
_L = 8
_F = 2
_T = 2 ** 17
_RES = [int(np.floor(16 * (1.5 ** l))) for l in range(_L)]
_P1 = np.uint32(2654435761)
_P2 = np.uint32(805459861)
_DH = 64

_NC = 2
_NS = 16
_NW = _NC * _NS
_C = 128


def _make_sc_encode(n, c, interpret=False):
    per_w = n // _NW
    nchunk = per_w // c
    g_per_c = c // 16
    nrow = _L * 8

    def body(x_hbm, bi_hbm, tabf_hbm, h_hbm,
             x_v0, bi_v0, idx_v0, vals_v0, h_v0, sem0,
             x_v1, bi_v1, idx_v1, vals_v1, h_v1, sem1):
        wid = lax.axis_index("s") * _NC + lax.axis_index("c")
        base_w = wid * per_w
        iota = lax.iota(jnp.int32, 16)
        bufs = ((x_v0, bi_v0, idx_v0, vals_v0, h_v0, sem0),
                (x_v1, bi_v1, idx_v1, vals_v1, h_v1, sem1))

        def load_idx_fire(k, buf):
            x_v, bi_v, idx_v, vals_v, _h, sem = buf
            base = base_w + k * c
            pltpu.sync_copy(x_hbm.at[pl.ds(base * 3, c * 3)], x_v)
            pltpu.sync_copy(bi_hbm.at[pl.ds(base, c)], bi_v)

            def idx_body(g, c2):
                o = g * 16
                i3 = (o + iota) * 3
                xv = [plsc.load_gather(x_v, [i3 + d]) for d in range(3)]
                bv = bi_v[pl.ds(o, 16)]
                for l in range(_L):
                    r1 = np.float32(_RES[l] - 1)
                    x0 = [(xv[d] * r1).astype(jnp.int32).astype(jnp.uint32)
                          for d in range(3)]
                    a0 = x0[0]
                    b0 = a0 + jnp.uint32(1)
                    a1 = x0[1] * _P1
                    b1 = a1 + _P1
                    a2 = x0[2] * _P2
                    b2 = a2 + _P2
                    lb = (bv + jnp.int32(l * _NB)) << 17
                    for cc in range(8):
                        hh = ((b0 if cc & 1 else a0)
                              ^ (b1 if cc & 2 else a1)
                              ^ (b2 if cc & 4 else a2)) & jnp.uint32(_T - 1)
                        e0 = hh.astype(jnp.int32) + lb
                        idx_v[pl.ds((l * 8 + cc) * c + o, 16)] = e0
                return c2

            lax.fori_loop(0, g_per_c, idx_body, 0)
            pltpu.async_copy(tabf_hbm.at[idx_v], vals_v, sem)

        def drain_acc_store(k, buf):
            x_v, _bi, idx_v, vals_v, h_v, sem = buf
            base = base_w + k * c
            pltpu.make_async_copy(tabf_hbm.at[idx_v], vals_v, sem).wait()

            def acc_body(g, c2):
                o = g * 16
                i3 = (o + iota) * 3
                xv = [plsc.load_gather(x_v, [i3 + d]) for d in range(3)]
                for l in range(_L):
                    r1 = np.float32(_RES[l] - 1)
                    xs = [xv[d] * r1 for d in range(3)]
                    w = [xs[d] - xs[d].astype(jnp.int32).astype(jnp.float32)
                         for d in range(3)]
                    u = [np.float32(1.0) - w[d] for d in range(3)]
                    acc0 = jnp.zeros((16,), jnp.float32)
                    acc1 = jnp.zeros((16,), jnp.float32)
                    for cc in range(8):
                        wc = ((w[0] if cc & 1 else u[0])
                              * (w[1] if cc & 2 else u[1])
                              * (w[2] if cc & 4 else u[2]))
                        r0 = (l * 8 + cc) * c + o + iota
                        v = plsc.load_gather(vals_v, [r0])
                        f0 = lax.bitcast_convert_type(
                            v << 16, jnp.float32)
                        f1 = lax.bitcast_convert_type(
                            v & jnp.int32(-65536), jnp.float32)
                        acc0 = acc0 + wc * f0
                        acc1 = acc1 + wc * f1
                    plsc.store_scatter(
                        h_v, [o + iota, jnp.full((16,), 2 * l, jnp.int32)],
                        acc0)
                    plsc.store_scatter(
                        h_v, [o + iota, jnp.full((16,), 2 * l + 1, jnp.int32)],
                        acc1)
                return c2

            lax.fori_loop(0, g_per_c, acc_body, 0)
            pltpu.sync_copy(h_v, h_hbm.at[pl.ds(base, c)])

        load_idx_fire(0, bufs[0])

        def pair_body(m, carry):
            k = 2 * m
            load_idx_fire(k + 1, bufs[1])
            drain_acc_store(k, bufs[0])
            load_idx_fire(k + 2, bufs[0])
            drain_acc_store(k + 1, bufs[1])
            return carry

        lax.fori_loop(0, nchunk // 2 - 1, pair_body, 0)
        k = nchunk - 2
        load_idx_fire(k + 1, bufs[1])
        drain_acc_store(k, bufs[0])
        drain_acc_store(k + 1, bufs[1])

    return pl.kernel(
        body,
        out_type=jax.ShapeDtypeStruct((n, 16), jnp.float32),
        mesh=plsc.VectorSubcoreMesh(
            core_axis_name="c", subcore_axis_name="s",
            num_cores=_NC, num_subcores=_NS),
        compiler_params=pltpu.CompilerParams(
            needs_layout_passes=False, use_tc_tiling_on_sc=False),
        scratch_types=[
            pltpu.VMEM((3 * c,), jnp.float32),
            pltpu.VMEM((c,), jnp.int32),
            pltpu.VMEM((nrow * c,), jnp.int32),
            pltpu.VMEM((nrow * c,), jnp.int32),
            pltpu.VMEM((c, 16), jnp.float32),
            pltpu.SemaphoreType.DMA,
            pltpu.VMEM((3 * c,), jnp.float32),
            pltpu.VMEM((c,), jnp.int32),
            pltpu.VMEM((nrow * c,), jnp.int32),
            pltpu.VMEM((nrow * c,), jnp.int32),
            pltpu.VMEM((c, 16), jnp.float32),
            pltpu.SemaphoreType.DMA,
        ],
        interpret=interpret,
    )


_sc_encode = _make_sc_encode(_N, _C)


def _mlp_body(h_ref, w1_ref, b1_ref, w2_ref, b2_ref, w3_ref, b3_ref, sdf_ref):
    h = h_ref[...]
    z = jnp.maximum(
        jnp.dot(h, w1_ref[...], preferred_element_type=jnp.float32)
        + b1_ref[...], 0.0)
    z = jnp.maximum(
        jnp.dot(z, w2_ref[...], preferred_element_type=jnp.float32)
        + b2_ref[...], 0.0)
    sdf_ref[...] = (
        jnp.dot(z, w3_ref[...], preferred_element_type=jnp.float32)
        + b3_ref[...])


def _mlp(h, W1, b1, W2, b2, W3, b3):
    bm = 4096
    wspec = lambda shape: pl.BlockSpec(shape, lambda i: (0, 0))
    return pl.pallas_call(
        _mlp_body,
        grid=(_N // bm,),
        in_specs=[
            pl.BlockSpec((bm, 16), lambda i: (i, 0)),
            wspec((16, _DH)), wspec((1, _DH)),
            wspec((_DH, _DH)), wspec((1, _DH)),
            wspec((_DH, 1)), wspec((1, 1)),
        ],
        out_specs=pl.BlockSpec((bm, 1), lambda i: (i, 0)),
        out_shape=jax.ShapeDtypeStruct((_N, 1), jnp.float32),
    )(h, W1, b1, W2, b2, W3, b3)


def kernel(x, block_inds, tables, W1, b1, W2, b2, W3, b3):
    x_flat = x.reshape(_N * 3)
    tabp = lax.bitcast_convert_type(
        tables.astype(jnp.bfloat16), jnp.int32).reshape(_L * _NB * _T)
    h = _sc_encode(x_flat, block_inds, tabp)
    sdf2 = _mlp(h, W1, b1.reshape(1, _DH), W2, b2.reshape(1, _DH),
                W3, b3.reshape(1, 1))
    return sdf2[:, 0], h
